# Initial kernel scaffold; baseline (speedup 1.0000x reference)
import jax
import jax.numpy as jnp
from jax.experimental import pallas as pl


def kernel(x, table):
    raise NotImplementedError("write your pallas kernel here")


# speedup vs baseline: 1.1112x; 1.1112x over previous
import functools

import jax
import jax.numpy as jnp
from jax import lax
from jax.experimental import pallas as pl
from jax.experimental.pallas import tpu as pltpu
from jax.experimental.pallas import tpu_sc as plsc

TOTAL = 16384 * 50
DIM = 32
NC, NS = 2, 16
NW = NC * NS
B_PER_W = TOTAL // NW
CHUNK = 128
N_CHUNK = B_PER_W // CHUNK
NBUF = 4


def _body(x_hbm, table_hbm, out_hbm, idx_v, rows_v, gsem, osem):
    wid = lax.axis_index("s") * NC + lax.axis_index("c")
    row0 = wid * N_CHUNK

    pltpu.sync_copy(x_hbm.at[pl.ds(row0, N_CHUNK)], idx_v)

    def start_gather(c, buf):
        pltpu.async_copy(table_hbm.at[idx_v.at[c]], rows_v.at[buf], gsem)

    def wait_gather(buf):
        pltpu.make_async_copy(table_hbm.at[idx_v.at[0]], rows_v.at[buf], gsem).wait()

    def start_out(c, buf):
        dst = out_hbm.at[pl.ds((row0 + c) * CHUNK, CHUNK)]
        pltpu.async_copy(rows_v.at[buf], dst, osem)

    def wait_out(buf):
        dst = out_hbm.at[pl.ds(0, CHUNK)]
        pltpu.make_async_copy(rows_v.at[buf], dst, osem).wait()

    for b in range(NBUF):
        start_gather(b, b)

    def loop(c, _):
        buf = lax.rem(c, NBUF)
        wait_gather(buf)
        start_out(c, buf)
        nc = c + NBUF

        @pl.when(nc < N_CHUNK)
        def _():
            wait_out(buf)
            start_gather(nc, buf)

        return 0

    lax.fori_loop(0, N_CHUNK, loop, 0)

    for b in range(NBUF):
        wait_out(b)


@functools.partial(jax.jit, static_argnames=())
def kernel(x, table):
    x_flat = x.reshape(TOTAL // CHUNK, CHUNK).astype(jnp.int32)
    mesh = plsc.VectorSubcoreMesh(core_axis_name="c", subcore_axis_name="s")
    out = pl.kernel(
        _body,
        out_type=jax.ShapeDtypeStruct((TOTAL, DIM), jnp.float32),
        mesh=mesh,
        scratch_types=[
            pltpu.VMEM((N_CHUNK, CHUNK), jnp.int32),
            pltpu.VMEM((NBUF, CHUNK, DIM), jnp.float32),
            pltpu.SemaphoreType.DMA,
            pltpu.SemaphoreType.DMA,
        ],
        compiler_params=pltpu.CompilerParams(use_tc_tiling_on_sc=False),
    )(x_flat, table)
    return out.reshape(x.shape[0], x.shape[1], DIM)


# speedup vs baseline: 1.1122x; 1.0010x over previous
import functools

import jax
import jax.numpy as jnp
from jax import lax
from jax.experimental import pallas as pl
from jax.experimental.pallas import tpu as pltpu
from jax.experimental.pallas import tpu_sc as plsc

TOTAL = 16384 * 50
DIM = 32
NC, NS = 2, 16
NW = NC * NS
B_PER_W = TOTAL // NW
CHUNK = 128
N_CHUNK = B_PER_W // CHUNK
RING = 8
DEPTH = 4


def _body(x_hbm, table_hbm, out_hbm, idx_v, rows_v, gsem, osem):
    wid = lax.axis_index("s") * NC + lax.axis_index("c")
    row0 = wid * N_CHUNK

    pltpu.sync_copy(x_hbm.at[pl.ds(row0, N_CHUNK)], idx_v)

    def start_gather(c, buf):
        pltpu.async_copy(table_hbm.at[idx_v.at[c]], rows_v.at[buf], gsem)

    def wait_gather(buf):
        pltpu.make_async_copy(table_hbm.at[idx_v.at[0]], rows_v.at[buf], gsem).wait()

    def start_out(c, buf):
        dst = out_hbm.at[pl.ds((row0 + c) * CHUNK, CHUNK)]
        pltpu.async_copy(rows_v.at[buf], dst, osem)

    def wait_out(buf):
        dst = out_hbm.at[pl.ds(0, CHUNK)]
        pltpu.make_async_copy(rows_v.at[buf], dst, osem).wait()

    for b in range(DEPTH):
        start_gather(b, b)

    def loop(c, _):
        buf = lax.rem(c, RING)
        wait_gather(buf)
        start_out(c, buf)
        nc = c + DEPTH

        @pl.when(nc < N_CHUNK)
        def _():
            @pl.when(nc >= RING)
            def _():
                wait_out(buf)

            start_gather(nc, lax.rem(nc, RING))

        return 0

    lax.fori_loop(0, N_CHUNK, loop, 0)

    for b in range(RING):
        wait_out(b)


@functools.partial(jax.jit, static_argnames=())
def kernel(x, table):
    x_flat = x.reshape(TOTAL // CHUNK, CHUNK).astype(jnp.int32)
    mesh = plsc.VectorSubcoreMesh(core_axis_name="c", subcore_axis_name="s")
    out = pl.kernel(
        _body,
        out_type=jax.ShapeDtypeStruct((TOTAL, DIM), jnp.float32),
        mesh=mesh,
        scratch_types=[
            pltpu.VMEM((N_CHUNK, CHUNK), jnp.int32),
            pltpu.VMEM((RING, CHUNK, DIM), jnp.float32),
            pltpu.SemaphoreType.DMA,
            pltpu.SemaphoreType.DMA,
        ],
        compiler_params=pltpu.CompilerParams(use_tc_tiling_on_sc=False),
    )(x_flat, table)
    return out.reshape(x.shape[0], x.shape[1], DIM)


# speedup vs baseline: 1.8098x; 1.6272x over previous
import functools

import jax
import jax.numpy as jnp
from jax import lax
from jax.experimental import pallas as pl
from jax.experimental.pallas import tpu as pltpu
from jax.experimental.pallas import tpu_sc as plsc
from jax.experimental.layout import Layout, with_layout_constraint

TOTAL = 16384 * 50
DIM = 32
NC, NS = 2, 16
NW = NC * NS
B_PER_W = TOTAL // NW
CHUNK = 128
N_CHUNK = B_PER_W // CHUNK
RING = 8
DEPTH = 4


def _body(x_hbm, table_hbm, out_hbm, idx_v, rows_v, gsem, osem):
    wid = lax.axis_index("s") * NC + lax.axis_index("c")
    row0 = wid * N_CHUNK

    pltpu.sync_copy(x_hbm.at[pl.ds(row0, N_CHUNK)], idx_v)

    def start_gather(c, buf):
        pltpu.async_copy(table_hbm.at[idx_v.at[c]], rows_v.at[buf], gsem)

    def wait_gather(buf):
        pltpu.make_async_copy(table_hbm.at[idx_v.at[0]], rows_v.at[buf], gsem).wait()

    def start_out(c, buf):
        dst = out_hbm.at[pl.ds((row0 + c) * CHUNK, CHUNK)]
        pltpu.async_copy(rows_v.at[buf], dst, osem)

    def wait_out(buf):
        dst = out_hbm.at[pl.ds(0, CHUNK)]
        pltpu.make_async_copy(rows_v.at[buf], dst, osem).wait()

    for b in range(DEPTH):
        start_gather(b, b)

    def loop(c, _):
        buf = lax.rem(c, RING)
        wait_gather(buf)
        start_out(c, buf)
        nc = c + DEPTH

        @pl.when(nc < N_CHUNK)
        def _():
            @pl.when(nc >= RING)
            def _():
                wait_out(buf)

            start_gather(nc, lax.rem(nc, RING))

        return 0

    lax.fori_loop(0, N_CHUNK, loop, 0)

    for b in range(RING):
        wait_out(b)


@functools.partial(jax.jit, static_argnames=())
def kernel(x, table):
    x_flat = x.reshape(TOTAL // CHUNK, CHUNK).astype(jnp.int32)
    mesh = plsc.VectorSubcoreMesh(core_axis_name="c", subcore_axis_name="s")
    out = pl.kernel(
        _body,
        out_type=jax.ShapeDtypeStruct((TOTAL, DIM), jnp.float32),
        mesh=mesh,
        scratch_types=[
            pltpu.VMEM((N_CHUNK, CHUNK), jnp.int32),
            pltpu.VMEM((RING, CHUNK, DIM), jnp.float32),
            pltpu.SemaphoreType.DMA,
            pltpu.SemaphoreType.DMA,
        ],
        compiler_params=pltpu.CompilerParams(use_tc_tiling_on_sc=False),
    )(x_flat, table)
    out = out.reshape(x.shape[0], x.shape[1], DIM)
    return with_layout_constraint(out, Layout(major_to_minor=(0, 1, 2)))


# speedup vs baseline: 2.0995x; 1.1601x over previous
import functools

import jax
import jax.numpy as jnp
from jax import lax
from jax.experimental import pallas as pl
from jax.experimental.pallas import tpu as pltpu
from jax.experimental.pallas import tpu_sc as plsc
from jax.experimental.layout import Layout, with_layout_constraint

TOTAL = 16384 * 50
DIM = 32
NC, NS = 2, 16
NW = NC * NS
B_PER_W = TOTAL // NW
CHUNK = 128
N_CHUNK = B_PER_W // CHUNK
RING = 8
DEPTH = 4


def _body(x_hbm, table_hbm, out_hbm, idx_v, rows_v, gsem, osem):
    wid = lax.axis_index("s") * NC + lax.axis_index("c")
    row0 = wid * N_CHUNK

    pltpu.sync_copy(x_hbm.at[pl.ds(row0, N_CHUNK)], idx_v)

    def start_gather(c, buf):
        pltpu.async_copy(table_hbm.at[idx_v.at[c]], rows_v.at[buf], gsem)

    def wait_gather(buf):
        pltpu.make_async_copy(table_hbm.at[idx_v.at[0]], rows_v.at[buf], gsem).wait()

    def start_out(c, buf):
        dst = out_hbm.at[pl.ds((row0 + c) * CHUNK, CHUNK)]
        pltpu.async_copy(rows_v.at[buf], dst, osem)

    def wait_out(buf):
        dst = out_hbm.at[pl.ds(0, CHUNK)]
        pltpu.make_async_copy(rows_v.at[buf], dst, osem).wait()

    for b in range(DEPTH):
        start_gather(b, b)

    def loop(c, _):
        buf = lax.rem(c, RING)
        wait_gather(buf)
        start_out(c, buf)
        nc = c + DEPTH

        @pl.when(nc < N_CHUNK)
        def _():
            @pl.when(nc >= RING)
            def _():
                wait_out(buf)

            start_gather(nc, lax.rem(nc, RING))

        return 0

    lax.fori_loop(0, N_CHUNK, loop, 0)

    for b in range(RING):
        wait_out(b)


def kernel(x, table):
    x_flat = x.reshape(TOTAL // CHUNK, CHUNK).astype(jnp.int32)
    mesh = plsc.VectorSubcoreMesh(core_axis_name="c", subcore_axis_name="s")
    out = pl.kernel(
        _body,
        out_type=jax.ShapeDtypeStruct((TOTAL, DIM), jnp.float32),
        mesh=mesh,
        scratch_types=[
            pltpu.VMEM((N_CHUNK, CHUNK), jnp.int32),
            pltpu.VMEM((RING, CHUNK, DIM), jnp.float32),
            pltpu.SemaphoreType.DMA,
            pltpu.SemaphoreType.DMA,
        ],
        compiler_params=pltpu.CompilerParams(use_tc_tiling_on_sc=False),
    )(x_flat, table)
    out = out.reshape(x.shape[0], x.shape[1], DIM)
    return with_layout_constraint(out, Layout(major_to_minor=(0, 1, 2)))
